# Initial kernel scaffold; baseline (speedup 1.0000x reference)
import jax
import jax.numpy as jnp
from jax.experimental import pallas as pl


def kernel(features, edge_index, W_self1, W_neigh1, b1, gamma1, beta1, W_self2, W_neigh2, b2, gamma2, beta2):
    raise NotImplementedError("write your pallas kernel here")


# speedup vs baseline: 11.9316x; 11.9316x over previous
import functools

import jax
import jax.numpy as jnp
from jax import lax
from jax.experimental import pallas as pl
from jax.experimental.pallas import tpu as pltpu
from jax.experimental.pallas import tpu_sc as plsc

N_NODES = 10000
N_EDGES = 320000
D = 128
BN_EPS = 1e-5

NC = 2
NS = 16
NW = NC * NS
CHUNK = 80
CPW = N_EDGES // (NW * CHUNK)
ROWS_PER_TILE = 640
NPAD = NS * ROWS_PER_TILE


def _sc_aggregate(with_deg):
    mesh = plsc.VectorSubcoreMesh(
        core_axis_name="c", subcore_axis_name="s",
        num_cores=NC, num_subcores=NS)

    out_type = [jax.ShapeDtypeStruct((NC, NPAD, D), jnp.float32)]
    scratch = [
        pltpu.VMEM_SHARED((NPAD, D), jnp.float32),
        pltpu.VMEM((CPW, CHUNK), jnp.int32),
        pltpu.VMEM((CPW, CHUNK), jnp.int32),
        pltpu.VMEM((CHUNK, D), jnp.float32),
        pltpu.VMEM((CHUNK, D), jnp.float32),
        pltpu.SemaphoreType.DMA,
        pltpu.SemaphoreType.DMA,
    ]
    if with_deg:
        out_type.append(jax.ShapeDtypeStruct((NC, NPAD), jnp.float32))
        scratch += [
            pltpu.VMEM_SHARED((NPAD,), jnp.float32),
            pltpu.VMEM((CHUNK,), jnp.float32),
            pltpu.VMEM((ROWS_PER_TILE,), jnp.float32),
        ]

    def body(xw_hbm, src_hbm, dst_hbm, parts_hbm, *rest):
        if with_deg:
            (degp_hbm, acc, srcv, dstv, rows0, rows1, sem0, sem1,
             dacc, ones, dz) = rest
        else:
            acc, srcv, dstv, rows0, rows1, sem0, sem1 = rest

        c = lax.axis_index("c")
        s = lax.axis_index("s")
        wid = s * NC + c
        row0 = s * ROWS_PER_TILE

        z16 = jnp.zeros((16,), jnp.float32)

        def zrow(i, carry):
            for j in range(D // 16):
                rows0[i, pl.ds(j * 16, 16)] = z16
            return carry
        lax.fori_loop(0, CHUNK, zrow, 0)
        for k in range(ROWS_PER_TILE // CHUNK):
            pltpu.sync_copy(rows0, acc.at[pl.ds(row0 + k * CHUNK, CHUNK)])

        if with_deg:
            o16 = jnp.ones((16,), jnp.float32)

            def zdeg(i, carry):
                dz[pl.ds(i * 16, 16)] = z16
                return carry
            lax.fori_loop(0, ROWS_PER_TILE // 16, zdeg, 0)
            pltpu.sync_copy(dz, dacc.at[pl.ds(row0, ROWS_PER_TILE)])

            def fones(i, carry):
                ones[pl.ds(i * 16, 16)] = o16
                return carry
            lax.fori_loop(0, CHUNK // 16, fones, 0)

        plsc.subcore_barrier()

        pltpu.sync_copy(src_hbm.at[pl.ds(wid * CPW, CPW)], srcv)
        pltpu.sync_copy(dst_hbm.at[pl.ds(wid * CPW, CPW)], dstv)

        def scatter(j, buf):
            pltpu.sync_copy(buf, acc.at[dstv.at[j]], add=True)
            if with_deg:
                pltpu.sync_copy(ones, dacc.at[dstv.at[j]], add=True)

        pltpu.async_copy(xw_hbm.at[srcv.at[0]], rows0, sem0)

        def step(i, carry):
            j0 = 2 * i
            pltpu.async_copy(xw_hbm.at[srcv.at[j0 + 1]], rows1, sem1)
            pltpu.make_async_copy(xw_hbm.at[srcv.at[j0]], rows0, sem0).wait()
            scatter(j0, rows0)
            pltpu.async_copy(xw_hbm.at[srcv.at[j0 + 2]], rows0, sem0)
            pltpu.make_async_copy(
                xw_hbm.at[srcv.at[j0 + 1]], rows1, sem1).wait()
            scatter(j0 + 1, rows1)
            return carry
        lax.fori_loop(0, (CPW - 1) // 2, step, 0)

        last = CPW - 1
        pltpu.make_async_copy(xw_hbm.at[srcv.at[last]], rows0, sem0).wait()
        scatter(last, rows0)

        plsc.subcore_barrier()

        pltpu.sync_copy(acc.at[pl.ds(row0, ROWS_PER_TILE)],
                        parts_hbm.at[c, pl.ds(row0, ROWS_PER_TILE)])
        if with_deg:
            pltpu.sync_copy(dacc.at[pl.ds(row0, ROWS_PER_TILE)],
                            degp_hbm.at[c, pl.ds(row0, ROWS_PER_TILE)])

    return pl.kernel(
        body, out_type=out_type, mesh=mesh, scratch_types=scratch,
        compiler_params=pltpu.CompilerParams(use_tc_tiling_on_sc=False))


def _tc_proj(x, w_self, w_neigh, b):
    def body(x_ref, ws_ref, wn_ref, b_ref, xs_ref, xw_ref):
        xb = x_ref[...]
        xs_ref[...] = jnp.dot(
            xb, ws_ref[...], preferred_element_type=jnp.float32) + b_ref[...]
        xw_ref[...] = jnp.dot(
            xb, wn_ref[...], preferred_element_type=jnp.float32)

    nblk = 10
    rb = N_NODES // nblk
    return pl.pallas_call(
        body,
        grid=(nblk,),
        in_specs=[
            pl.BlockSpec((rb, D), lambda i: (i, 0)),
            pl.BlockSpec((D, D), lambda i: (0, 0)),
            pl.BlockSpec((D, D), lambda i: (0, 0)),
            pl.BlockSpec((1, D), lambda i: (0, 0)),
        ],
        out_specs=[
            pl.BlockSpec((rb, D), lambda i: (i, 0)),
            pl.BlockSpec((rb, D), lambda i: (i, 0)),
        ],
        out_shape=[
            jax.ShapeDtypeStruct((N_NODES, D), jnp.float32),
            jax.ShapeDtypeStruct((N_NODES, D), jnp.float32),
        ],
    )(x, w_self, w_neigh, b.reshape(1, D))


def _combine_bn(xs_ref, p_ref, dg_ref):
    p = p_ref[0, :N_NODES, :] + p_ref[1, :N_NODES, :]
    deg = dg_ref[0, :N_NODES, :] + dg_ref[1, :N_NODES, :]
    h = xs_ref[...] + p / jnp.maximum(deg, 1.0)
    mu = jnp.mean(h, axis=0, keepdims=True)
    var = jnp.mean((h - mu) ** 2, axis=0, keepdims=True)
    return h, mu, var


def _tc_mid(xs1, parts1, degp, gamma1, beta1, w_self2, w_neigh2, b2):
    def body(xs_ref, p_ref, dg_ref, g_ref, bt_ref, ws_ref, wn_ref, b2_ref,
             xs2_ref, xw2_ref):
        h, mu, var = _combine_bn(xs_ref, p_ref, dg_ref)
        h = g_ref[...] * (h - mu) * lax.rsqrt(var + BN_EPS) + bt_ref[...]
        h = jnp.maximum(h, 0.0)
        xs2_ref[...] = jnp.dot(
            h, ws_ref[...], preferred_element_type=jnp.float32) + b2_ref[...]
        xw2_ref[...] = jnp.dot(
            h, wn_ref[...], preferred_element_type=jnp.float32)

    return pl.pallas_call(
        body,
        out_shape=[
            jax.ShapeDtypeStruct((N_NODES, D), jnp.float32),
            jax.ShapeDtypeStruct((N_NODES, D), jnp.float32),
        ],
    )(xs1, parts1, degp, gamma1.reshape(1, D), beta1.reshape(1, D),
      w_self2, w_neigh2, b2.reshape(1, D))


def _tc_final(xs2, parts2, degp, gamma2, beta2):
    def body(xs_ref, p_ref, dg_ref, g_ref, bt_ref, out_ref):
        h, mu, var = _combine_bn(xs_ref, p_ref, dg_ref)
        out_ref[...] = (g_ref[...] * (h - mu) * lax.rsqrt(var + BN_EPS)
                        + bt_ref[...])

    return pl.pallas_call(
        body,
        out_shape=jax.ShapeDtypeStruct((N_NODES, D), jnp.float32),
    )(xs2, parts2, degp, gamma2.reshape(1, D), beta2.reshape(1, D))


def kernel(features, edge_index, W_self1, W_neigh1, b1, gamma1, beta1,
           W_self2, W_neigh2, b2, gamma2, beta2):
    ei = edge_index.astype(jnp.int32)
    src2d = ei[0].reshape(NW * CPW, CHUNK)
    dst2d = ei[1].reshape(NW * CPW, CHUNK)

    xs1, xw1 = _tc_proj(features, W_self1, W_neigh1, b1)
    parts1, degp = _sc_aggregate(with_deg=True)(xw1, src2d, dst2d)
    degp3 = degp.reshape(NC, NPAD, 1)
    xs2, xw2 = _tc_mid(xs1, parts1, degp3, gamma1, beta1,
                       W_self2, W_neigh2, b2)
    (parts2,) = _sc_aggregate(with_deg=False)(xw2, src2d, dst2d)
    return _tc_final(xs2, parts2, degp3, gamma2, beta2)


# speedup vs baseline: 13.5339x; 1.1343x over previous
import functools

import jax
import jax.numpy as jnp
from jax import lax
from jax.experimental import pallas as pl
from jax.experimental.pallas import tpu as pltpu
from jax.experimental.pallas import tpu_sc as plsc

N_NODES = 10000
N_EDGES = 320000
D = 128
BN_EPS = 1e-5

NC = 2
NS = 16
NW = NC * NS
CHUNK = 80
CPW = N_EDGES // (NW * CHUNK)
ROWS_PER_TILE = 640
NPAD = NS * ROWS_PER_TILE


def _sc_aggregate(with_deg):
    mesh = plsc.VectorSubcoreMesh(
        core_axis_name="c", subcore_axis_name="s",
        num_cores=NC, num_subcores=NS)

    out_type = [jax.ShapeDtypeStruct((NC, NPAD, D), jnp.float32)]
    scratch = [
        pltpu.VMEM_SHARED((NPAD, D), jnp.float32),
        pltpu.VMEM((CPW, CHUNK), jnp.int32),
        pltpu.VMEM((CHUNK, D), jnp.float32),
        pltpu.VMEM((CHUNK, D), jnp.float32),
        pltpu.VMEM((CHUNK, D), jnp.float32),
        pltpu.VMEM((CHUNK,), jnp.int32),
        pltpu.VMEM((CHUNK,), jnp.int32),
        pltpu.VMEM((CHUNK,), jnp.int32),
        pltpu.SemaphoreType.DMA,
        pltpu.SemaphoreType.DMA,
        pltpu.SemaphoreType.DMA,
        pltpu.SemaphoreType.DMA,
        pltpu.SemaphoreType.DMA,
        pltpu.SemaphoreType.DMA,
        pltpu.SemaphoreType.DMA,
        pltpu.SemaphoreType.DMA,
        pltpu.SemaphoreType.DMA,
    ]
    if with_deg:
        out_type.append(jax.ShapeDtypeStruct((NC, NPAD), jnp.float32))
        scratch += [
            pltpu.VMEM_SHARED((NPAD,), jnp.float32),
            pltpu.VMEM((CHUNK,), jnp.float32),
            pltpu.VMEM((ROWS_PER_TILE,), jnp.float32),
        ]

    def body(xw_hbm, src_hbm, dst_hbm, parts_hbm, *rest):
        if with_deg:
            (degp_hbm, acc, srcv, r0, r1, r2, d0, d1, d2,
             g0, g1, g2, s0, s1, s2, i0, i1, i2, dacc, ones, dz) = rest
        else:
            (acc, srcv, r0, r1, r2, d0, d1, d2,
             g0, g1, g2, s0, s1, s2, i0, i1, i2) = rest
        rows = (r0, r1, r2)
        dsti = (d0, d1, d2)
        semg = (g0, g1, g2)
        sems = (s0, s1, s2)
        semi = (i0, i1, i2)

        c = lax.axis_index("c")
        s = lax.axis_index("s")
        wid = s * NC + c
        row0 = s * ROWS_PER_TILE
        ebase = wid * CPW

        z16 = jnp.zeros((16,), jnp.float32)

        def zrow(i, carry):
            for j in range(D // 16):
                r0[i, pl.ds(j * 16, 16)] = z16
            return carry
        lax.fori_loop(0, CHUNK, zrow, 0)
        for k in range(ROWS_PER_TILE // CHUNK):
            pltpu.sync_copy(r0, acc.at[pl.ds(row0 + k * CHUNK, CHUNK)])

        if with_deg:
            o16 = jnp.ones((16,), jnp.float32)

            def zdeg(i, carry):
                dz[pl.ds(i * 16, 16)] = z16
                return carry
            lax.fori_loop(0, ROWS_PER_TILE // 16, zdeg, 0)
            pltpu.sync_copy(dz, dacc.at[pl.ds(row0, ROWS_PER_TILE)])

            def fones(i, carry):
                ones[pl.ds(i * 16, 16)] = o16
                return carry
            lax.fori_loop(0, CHUNK // 16, fones, 0)

        plsc.subcore_barrier()

        pltpu.sync_copy(src_hbm.at[pl.ds(ebase, CPW)], srcv)

        def fetch(j, b):
            pltpu.async_copy(dst_hbm.at[ebase + j], dsti[b], semi[b])
            pltpu.async_copy(xw_hbm.at[srcv.at[j]], rows[b], semg[b])

        def wait_fetch(j, b):
            pltpu.make_async_copy(dst_hbm.at[ebase + j], dsti[b],
                                  semi[b]).wait()
            pltpu.make_async_copy(xw_hbm.at[srcv.at[j]], rows[b],
                                  semg[b]).wait()

        def scatter(b):
            pltpu.async_copy(rows[b], acc.at[dsti[b]], sems[b], add=True)
            if with_deg:
                pltpu.sync_copy(ones, dacc.at[dsti[b]], add=True)

        def wait_scatter(b):
            pltpu.make_async_copy(rows[b], acc.at[dsti[b]], sems[b]).wait()

        fetch(0, 0)
        fetch(1, 1)

        def step(i, carry):
            j = 3 * i
            wait_fetch(j, 0)
            scatter(0)

            @pl.when(i > 0)
            def _():
                wait_scatter(2)
            fetch(j + 2, 2)

            wait_fetch(j + 1, 1)
            scatter(1)
            wait_scatter(0)
            fetch(j + 3, 0)

            wait_fetch(j + 2, 2)
            scatter(2)
            wait_scatter(1)
            fetch(j + 4, 1)
            return carry
        lax.fori_loop(0, (CPW - 2) // 3, step, 0)

        wait_fetch(CPW - 2, 0)
        scatter(0)
        wait_fetch(CPW - 1, 1)
        scatter(1)
        wait_scatter(2)
        wait_scatter(0)
        wait_scatter(1)

        plsc.subcore_barrier()

        pltpu.sync_copy(acc.at[pl.ds(row0, ROWS_PER_TILE)],
                        parts_hbm.at[c, pl.ds(row0, ROWS_PER_TILE)])
        if with_deg:
            pltpu.sync_copy(dacc.at[pl.ds(row0, ROWS_PER_TILE)],
                            degp_hbm.at[c, pl.ds(row0, ROWS_PER_TILE)])

    return pl.kernel(
        body, out_type=out_type, mesh=mesh, scratch_types=scratch,
        compiler_params=pltpu.CompilerParams(use_tc_tiling_on_sc=False))


def _tc_proj(x, w_self, w_neigh, b):
    def body(x_ref, ws_ref, wn_ref, b_ref, xs_ref, xw_ref):
        xb = x_ref[...]
        xs_ref[...] = jnp.dot(
            xb, ws_ref[...], preferred_element_type=jnp.float32) + b_ref[...]
        xw_ref[...] = jnp.dot(
            xb, wn_ref[...], preferred_element_type=jnp.float32)

    nblk = 10
    rb = N_NODES // nblk
    return pl.pallas_call(
        body,
        grid=(nblk,),
        in_specs=[
            pl.BlockSpec((rb, D), lambda i: (i, 0)),
            pl.BlockSpec((D, D), lambda i: (0, 0)),
            pl.BlockSpec((D, D), lambda i: (0, 0)),
            pl.BlockSpec((1, D), lambda i: (0, 0)),
        ],
        out_specs=[
            pl.BlockSpec((rb, D), lambda i: (i, 0)),
            pl.BlockSpec((rb, D), lambda i: (i, 0)),
        ],
        out_shape=[
            jax.ShapeDtypeStruct((N_NODES, D), jnp.float32),
            jax.ShapeDtypeStruct((N_NODES, D), jnp.float32),
        ],
    )(x, w_self, w_neigh, b.reshape(1, D))


def _combine_bn(xs_ref, p_ref, dg_ref):
    p = p_ref[0, :N_NODES, :] + p_ref[1, :N_NODES, :]
    deg = dg_ref[0, :N_NODES, :] + dg_ref[1, :N_NODES, :]
    h = xs_ref[...] + p / jnp.maximum(deg, 1.0)
    mu = jnp.mean(h, axis=0, keepdims=True)
    var = jnp.mean((h - mu) ** 2, axis=0, keepdims=True)
    return h, mu, var


def _tc_mid(xs1, parts1, degp, gamma1, beta1, w_self2, w_neigh2, b2):
    def body(xs_ref, p_ref, dg_ref, g_ref, bt_ref, ws_ref, wn_ref, b2_ref,
             xs2_ref, xw2_ref):
        h, mu, var = _combine_bn(xs_ref, p_ref, dg_ref)
        h = g_ref[...] * (h - mu) * lax.rsqrt(var + BN_EPS) + bt_ref[...]
        h = jnp.maximum(h, 0.0)
        xs2_ref[...] = jnp.dot(
            h, ws_ref[...], preferred_element_type=jnp.float32) + b2_ref[...]
        xw2_ref[...] = jnp.dot(
            h, wn_ref[...], preferred_element_type=jnp.float32)

    return pl.pallas_call(
        body,
        out_shape=[
            jax.ShapeDtypeStruct((N_NODES, D), jnp.float32),
            jax.ShapeDtypeStruct((N_NODES, D), jnp.float32),
        ],
    )(xs1, parts1, degp, gamma1.reshape(1, D), beta1.reshape(1, D),
      w_self2, w_neigh2, b2.reshape(1, D))


def _tc_final(xs2, parts2, degp, gamma2, beta2):
    def body(xs_ref, p_ref, dg_ref, g_ref, bt_ref, out_ref):
        h, mu, var = _combine_bn(xs_ref, p_ref, dg_ref)
        out_ref[...] = (g_ref[...] * (h - mu) * lax.rsqrt(var + BN_EPS)
                        + bt_ref[...])

    return pl.pallas_call(
        body,
        out_shape=jax.ShapeDtypeStruct((N_NODES, D), jnp.float32),
    )(xs2, parts2, degp, gamma2.reshape(1, D), beta2.reshape(1, D))


def kernel(features, edge_index, W_self1, W_neigh1, b1, gamma1, beta1,
           W_self2, W_neigh2, b2, gamma2, beta2):
    ei = edge_index.astype(jnp.int32)
    src2d = ei[0].reshape(NW * CPW, CHUNK)
    dst2d = ei[1].reshape(NW * CPW, CHUNK)

    xs1, xw1 = _tc_proj(features, W_self1, W_neigh1, b1)
    parts1, degp = _sc_aggregate(with_deg=True)(xw1, src2d, dst2d)
    degp3 = degp.reshape(NC, NPAD, 1)
    xs2, xw2 = _tc_mid(xs1, parts1, degp3, gamma1, beta1,
                       W_self2, W_neigh2, b2)
    (parts2,) = _sc_aggregate(with_deg=False)(xw2, src2d, dst2d)
    return _tc_final(xs2, parts2, degp3, gamma2, beta2)


# speedup vs baseline: 14.3735x; 1.0620x over previous
import functools

import jax
import jax.numpy as jnp
from jax import lax
from jax.experimental import pallas as pl
from jax.experimental.pallas import tpu as pltpu
from jax.experimental.pallas import tpu_sc as plsc

N_NODES = 10000
N_EDGES = 320000
D = 128
BN_EPS = 1e-5

NC = 2
NS = 16
NW = NC * NS
CHUNK = 80
CPW = N_EDGES // (NW * CHUNK)
ROWS_PER_TILE = 640
NPAD = NS * ROWS_PER_TILE


def _sc_aggregate(with_deg):
    mesh = plsc.VectorSubcoreMesh(
        core_axis_name="c", subcore_axis_name="s",
        num_cores=NC, num_subcores=NS)

    out_type = [jax.ShapeDtypeStruct((NC, NPAD, D), jnp.float32)]
    scratch = [
        pltpu.VMEM_SHARED((NPAD, D), jnp.float32),
        pltpu.VMEM((CPW, CHUNK), jnp.int32),
        pltpu.VMEM((CHUNK, D), jnp.float32),
        pltpu.VMEM((CHUNK, D), jnp.float32),
        pltpu.VMEM((CHUNK, D), jnp.float32),
        pltpu.VMEM((CHUNK,), jnp.int32),
        pltpu.VMEM((CHUNK,), jnp.int32),
        pltpu.VMEM((CHUNK,), jnp.int32),
        pltpu.SemaphoreType.DMA,
        pltpu.SemaphoreType.DMA,
        pltpu.SemaphoreType.DMA,
        pltpu.SemaphoreType.DMA,
        pltpu.SemaphoreType.DMA,
        pltpu.SemaphoreType.DMA,
        pltpu.SemaphoreType.DMA,
        pltpu.SemaphoreType.DMA,
        pltpu.SemaphoreType.DMA,
    ]
    if with_deg:
        out_type.append(jax.ShapeDtypeStruct((NC, NPAD), jnp.float32))
        scratch += [
            pltpu.VMEM_SHARED((NPAD,), jnp.float32),
            pltpu.VMEM((CHUNK,), jnp.float32),
            pltpu.VMEM((ROWS_PER_TILE,), jnp.float32),
            pltpu.SemaphoreType.DMA,
            pltpu.SemaphoreType.DMA,
            pltpu.SemaphoreType.DMA,
        ]

    def body(xw_hbm, src_hbm, dst_hbm, parts_hbm, *rest):
        if with_deg:
            (degp_hbm, acc, srcv, r0, r1, r2, d0, d1, d2,
             g0, g1, g2, s0, s1, s2, i0, i1, i2,
             dacc, ones, dz, e0, e1, e2) = rest
            semd = (e0, e1, e2)
        else:
            (acc, srcv, r0, r1, r2, d0, d1, d2,
             g0, g1, g2, s0, s1, s2, i0, i1, i2) = rest
        rows = (r0, r1, r2)
        dsti = (d0, d1, d2)
        semg = (g0, g1, g2)
        sems = (s0, s1, s2)
        semi = (i0, i1, i2)

        c = lax.axis_index("c")
        s = lax.axis_index("s")
        wid = s * NC + c
        row0 = s * ROWS_PER_TILE
        ebase = wid * CPW

        z16 = jnp.zeros((16,), jnp.float32)

        def zrow(i, carry):
            for j in range(D // 16):
                r0[i, pl.ds(j * 16, 16)] = z16
            return carry
        lax.fori_loop(0, CHUNK, zrow, 0)
        for k in range(ROWS_PER_TILE // CHUNK):
            pltpu.sync_copy(r0, acc.at[pl.ds(row0 + k * CHUNK, CHUNK)])

        if with_deg:
            o16 = jnp.ones((16,), jnp.float32)

            def zdeg(i, carry):
                dz[pl.ds(i * 16, 16)] = z16
                return carry
            lax.fori_loop(0, ROWS_PER_TILE // 16, zdeg, 0)
            pltpu.sync_copy(dz, dacc.at[pl.ds(row0, ROWS_PER_TILE)])

            def fones(i, carry):
                ones[pl.ds(i * 16, 16)] = o16
                return carry
            lax.fori_loop(0, CHUNK // 16, fones, 0)

        plsc.subcore_barrier()

        pltpu.sync_copy(src_hbm.at[pl.ds(ebase, CPW)], srcv)

        def fetch(j, b):
            pltpu.async_copy(dst_hbm.at[ebase + j], dsti[b], semi[b])
            pltpu.async_copy(xw_hbm.at[srcv.at[j]], rows[b], semg[b])

        def wait_fetch(j, b):
            pltpu.make_async_copy(dst_hbm.at[ebase + j], dsti[b],
                                  semi[b]).wait()
            pltpu.make_async_copy(xw_hbm.at[srcv.at[j]], rows[b],
                                  semg[b]).wait()

        def scatter(b):
            pltpu.async_copy(rows[b], acc.at[dsti[b]], sems[b], add=True)
            if with_deg:
                pltpu.async_copy(ones, dacc.at[dsti[b]], semd[b], add=True)

        def wait_scatter(b):
            pltpu.make_async_copy(rows[b], acc.at[dsti[b]], sems[b]).wait()
            if with_deg:
                pltpu.make_async_copy(ones, dacc.at[dsti[b]],
                                      semd[b]).wait()

        fetch(0, 0)
        fetch(1, 1)

        def step(i, carry):
            j = 3 * i
            wait_fetch(j, 0)
            scatter(0)

            @pl.when(i > 0)
            def _():
                wait_scatter(2)
            fetch(j + 2, 2)

            wait_fetch(j + 1, 1)
            scatter(1)
            wait_scatter(0)
            fetch(j + 3, 0)

            wait_fetch(j + 2, 2)
            scatter(2)
            wait_scatter(1)
            fetch(j + 4, 1)
            return carry
        lax.fori_loop(0, (CPW - 2) // 3, step, 0)

        wait_fetch(CPW - 2, 0)
        scatter(0)
        wait_fetch(CPW - 1, 1)
        scatter(1)
        wait_scatter(2)
        wait_scatter(0)
        wait_scatter(1)

        plsc.subcore_barrier()

        pltpu.sync_copy(acc.at[pl.ds(row0, ROWS_PER_TILE)],
                        parts_hbm.at[c, pl.ds(row0, ROWS_PER_TILE)])
        if with_deg:
            pltpu.sync_copy(dacc.at[pl.ds(row0, ROWS_PER_TILE)],
                            degp_hbm.at[c, pl.ds(row0, ROWS_PER_TILE)])

    return pl.kernel(
        body, out_type=out_type, mesh=mesh, scratch_types=scratch,
        compiler_params=pltpu.CompilerParams(use_tc_tiling_on_sc=False))


def _sage_layer(x_ref, p_ref, dg_ref, ws_ref, wn_ref, b_ref, g_ref, bt_ref):
    p = p_ref[0, :N_NODES, :] + p_ref[1, :N_NODES, :]
    deg = dg_ref[0, :N_NODES, :] + dg_ref[1, :N_NODES, :]
    hn = p / jnp.maximum(deg, 1.0)
    h = (jnp.dot(x_ref[...], ws_ref[...],
                 preferred_element_type=jnp.float32) + b_ref[...]
         + jnp.dot(hn, wn_ref[...], preferred_element_type=jnp.float32))
    mu = jnp.mean(h, axis=0, keepdims=True)
    var = jnp.mean((h - mu) ** 2, axis=0, keepdims=True)
    return g_ref[...] * (h - mu) * lax.rsqrt(var + BN_EPS) + bt_ref[...]


def _tc_mid(x, parts1, degp, w_self1, w_neigh1, b1, gamma1, beta1):
    def body(x_ref, p_ref, dg_ref, ws_ref, wn_ref, b_ref, g_ref, bt_ref,
             h1_ref):
        h = _sage_layer(x_ref, p_ref, dg_ref, ws_ref, wn_ref, b_ref,
                        g_ref, bt_ref)
        h1_ref[...] = jnp.maximum(h, 0.0)

    return pl.pallas_call(
        body,
        out_shape=jax.ShapeDtypeStruct((N_NODES, D), jnp.float32),
    )(x, parts1, degp, w_self1, w_neigh1, b1.reshape(1, D),
      gamma1.reshape(1, D), beta1.reshape(1, D))


def _tc_final(h1, parts2, degp, w_self2, w_neigh2, b2, gamma2, beta2):
    def body(x_ref, p_ref, dg_ref, ws_ref, wn_ref, b_ref, g_ref, bt_ref,
             out_ref):
        out_ref[...] = _sage_layer(x_ref, p_ref, dg_ref, ws_ref, wn_ref,
                                   b_ref, g_ref, bt_ref)

    return pl.pallas_call(
        body,
        out_shape=jax.ShapeDtypeStruct((N_NODES, D), jnp.float32),
    )(h1, parts2, degp, w_self2, w_neigh2, b2.reshape(1, D),
      gamma2.reshape(1, D), beta2.reshape(1, D))


def kernel(features, edge_index, W_self1, W_neigh1, b1, gamma1, beta1,
           W_self2, W_neigh2, b2, gamma2, beta2):
    ei = edge_index.astype(jnp.int32)
    src2d = ei[0].reshape(NW * CPW, CHUNK)
    dst2d = ei[1].reshape(NW * CPW, CHUNK)

    parts1, degp = _sc_aggregate(with_deg=True)(features, src2d, dst2d)
    degp3 = degp.reshape(NC, NPAD, 1)
    h1 = _tc_mid(features, parts1, degp3, W_self1, W_neigh1, b1,
                 gamma1, beta1)
    (parts2,) = _sc_aggregate(with_deg=False)(h1, src2d, dst2d)
    return _tc_final(h1, parts2, degp3, W_self2, W_neigh2, b2,
                     gamma2, beta2)


# speedup vs baseline: 15.9734x; 1.1113x over previous
import functools

import jax
import jax.numpy as jnp
from jax import lax
from jax.experimental import pallas as pl
from jax.experimental.pallas import tpu as pltpu
from jax.experimental.pallas import tpu_sc as plsc

N_NODES = 10000
N_EDGES = 320000
D = 128
BN_EPS = 1e-5

NC = 2
NS = 16
NW = NC * NS
CHUNK = 80
CPW = N_EDGES // (NW * CHUNK)
ROWS_PER_TILE = 640
NPAD = NS * ROWS_PER_TILE


def _sc_aggregate(with_deg):
    mesh = plsc.VectorSubcoreMesh(
        core_axis_name="c", subcore_axis_name="s",
        num_cores=NC, num_subcores=NS)

    out_type = [jax.ShapeDtypeStruct((NC, NPAD, D), jnp.float32)]
    scratch = [
        pltpu.VMEM_SHARED((NPAD, D), jnp.float32),
        pltpu.VMEM((CPW * CHUNK,), jnp.int32),
        pltpu.VMEM((CHUNK, D), jnp.float32),
        pltpu.VMEM((CHUNK, D), jnp.float32),
        pltpu.VMEM((CHUNK, D), jnp.float32),
        pltpu.VMEM((CHUNK,), jnp.int32),
        pltpu.VMEM((CHUNK,), jnp.int32),
        pltpu.VMEM((CHUNK,), jnp.int32),
        pltpu.SemaphoreType.DMA,
        pltpu.SemaphoreType.DMA,
        pltpu.SemaphoreType.DMA,
        pltpu.SemaphoreType.DMA,
        pltpu.SemaphoreType.DMA,
        pltpu.SemaphoreType.DMA,
        pltpu.SemaphoreType.DMA,
        pltpu.SemaphoreType.DMA,
        pltpu.SemaphoreType.DMA,
    ]
    if with_deg:
        out_type.append(jax.ShapeDtypeStruct((NC, NPAD), jnp.float32))
        scratch += [
            pltpu.VMEM_SHARED((NPAD,), jnp.float32),
            pltpu.VMEM((CHUNK,), jnp.float32),
            pltpu.VMEM((ROWS_PER_TILE,), jnp.float32),
            pltpu.SemaphoreType.DMA,
            pltpu.SemaphoreType.DMA,
            pltpu.SemaphoreType.DMA,
        ]

    def body(xw_hbm, ei_hbm, parts_hbm, *rest):
        if with_deg:
            (degp_hbm, acc, srcv, r0, r1, r2, d0, d1, d2,
             g0, g1, g2, s0, s1, s2, i0, i1, i2,
             dacc, ones, dz, e0, e1, e2) = rest
            semd = (e0, e1, e2)
        else:
            (acc, srcv, r0, r1, r2, d0, d1, d2,
             g0, g1, g2, s0, s1, s2, i0, i1, i2) = rest
        rows = (r0, r1, r2)
        dsti = (d0, d1, d2)
        semg = (g0, g1, g2)
        sems = (s0, s1, s2)
        semi = (i0, i1, i2)

        c = lax.axis_index("c")
        s = lax.axis_index("s")
        wid = s * NC + c
        row0 = s * ROWS_PER_TILE
        ebase = wid * CPW * CHUNK

        z16 = jnp.zeros((16,), jnp.float32)

        def zrow(i, carry):
            for j in range(D // 16):
                r0[i, pl.ds(j * 16, 16)] = z16
            return carry
        lax.fori_loop(0, CHUNK, zrow, 0)
        for k in range(ROWS_PER_TILE // CHUNK):
            pltpu.sync_copy(r0, acc.at[pl.ds(row0 + k * CHUNK, CHUNK)])

        if with_deg:
            o16 = jnp.ones((16,), jnp.float32)

            def zdeg(i, carry):
                dz[pl.ds(i * 16, 16)] = z16
                return carry
            lax.fori_loop(0, ROWS_PER_TILE // 16, zdeg, 0)
            pltpu.sync_copy(dz, dacc.at[pl.ds(row0, ROWS_PER_TILE)])

            def fones(i, carry):
                ones[pl.ds(i * 16, 16)] = o16
                return carry
            lax.fori_loop(0, CHUNK // 16, fones, 0)

        plsc.subcore_barrier()

        pltpu.sync_copy(ei_hbm.at[0, pl.ds(ebase, CPW * CHUNK)], srcv)

        def fetch(j, b):
            pltpu.async_copy(ei_hbm.at[1, pl.ds(ebase + j * CHUNK, CHUNK)],
                             dsti[b], semi[b])
            pltpu.async_copy(xw_hbm.at[srcv.at[pl.ds(j * CHUNK, CHUNK)]],
                             rows[b], semg[b])

        def wait_fetch(j, b):
            pltpu.make_async_copy(
                ei_hbm.at[1, pl.ds(ebase + j * CHUNK, CHUNK)],
                dsti[b], semi[b]).wait()
            pltpu.make_async_copy(xw_hbm.at[srcv.at[pl.ds(j * CHUNK, CHUNK)]],
                                  rows[b], semg[b]).wait()

        def scatter(b):
            pltpu.async_copy(rows[b], acc.at[dsti[b]], sems[b], add=True)
            if with_deg:
                pltpu.async_copy(ones, dacc.at[dsti[b]], semd[b], add=True)

        def wait_scatter(b):
            pltpu.make_async_copy(rows[b], acc.at[dsti[b]], sems[b]).wait()
            if with_deg:
                pltpu.make_async_copy(ones, dacc.at[dsti[b]],
                                      semd[b]).wait()

        fetch(0, 0)
        fetch(1, 1)

        def step(i, carry):
            j = 3 * i
            wait_fetch(j, 0)
            scatter(0)

            @pl.when(i > 0)
            def _():
                wait_scatter(2)
            fetch(j + 2, 2)

            wait_fetch(j + 1, 1)
            scatter(1)
            wait_scatter(0)
            fetch(j + 3, 0)

            wait_fetch(j + 2, 2)
            scatter(2)
            wait_scatter(1)
            fetch(j + 4, 1)
            return carry
        lax.fori_loop(0, (CPW - 2) // 3, step, 0)

        wait_fetch(CPW - 2, 0)
        scatter(0)
        wait_fetch(CPW - 1, 1)
        scatter(1)
        wait_scatter(2)
        wait_scatter(0)
        wait_scatter(1)

        plsc.subcore_barrier()

        pltpu.sync_copy(acc.at[pl.ds(row0, ROWS_PER_TILE)],
                        parts_hbm.at[c, pl.ds(row0, ROWS_PER_TILE)])
        if with_deg:
            pltpu.sync_copy(dacc.at[pl.ds(row0, ROWS_PER_TILE)],
                            degp_hbm.at[c, pl.ds(row0, ROWS_PER_TILE)])

    return pl.kernel(
        body, out_type=out_type, mesh=mesh, scratch_types=scratch,
        compiler_params=pltpu.CompilerParams(use_tc_tiling_on_sc=False))


def _sage_layer(x_ref, p_ref, dg_ref, ws_ref, wn_ref, b_ref, g_ref, bt_ref):
    p = p_ref[0, :N_NODES, :] + p_ref[1, :N_NODES, :]
    deg = dg_ref[0, :N_NODES] + dg_ref[1, :N_NODES]
    hn = p / jnp.maximum(deg, 1.0)[:, None]
    h = (jnp.dot(x_ref[...], ws_ref[...],
                 preferred_element_type=jnp.float32) + b_ref[...]
         + jnp.dot(hn, wn_ref[...], preferred_element_type=jnp.float32))
    mu = jnp.mean(h, axis=0, keepdims=True)
    var = jnp.mean((h - mu) ** 2, axis=0, keepdims=True)
    return g_ref[...] * (h - mu) * lax.rsqrt(var + BN_EPS) + bt_ref[...]


def _tc_mid(x, parts1, degp, w_self1, w_neigh1, b1, gamma1, beta1):
    def body(x_ref, p_ref, dg_ref, ws_ref, wn_ref, b_ref, g_ref, bt_ref,
             h1_ref):
        h = _sage_layer(x_ref, p_ref, dg_ref, ws_ref, wn_ref, b_ref,
                        g_ref, bt_ref)
        h1_ref[...] = jnp.maximum(h, 0.0)

    return pl.pallas_call(
        body,
        out_shape=jax.ShapeDtypeStruct((N_NODES, D), jnp.float32),
    )(x, parts1, degp, w_self1, w_neigh1, b1.reshape(1, D),
      gamma1.reshape(1, D), beta1.reshape(1, D))


def _tc_final(h1, parts2, degp, w_self2, w_neigh2, b2, gamma2, beta2):
    def body(x_ref, p_ref, dg_ref, ws_ref, wn_ref, b_ref, g_ref, bt_ref,
             out_ref):
        out_ref[...] = _sage_layer(x_ref, p_ref, dg_ref, ws_ref, wn_ref,
                                   b_ref, g_ref, bt_ref)

    return pl.pallas_call(
        body,
        out_shape=jax.ShapeDtypeStruct((N_NODES, D), jnp.float32),
    )(h1, parts2, degp, w_self2, w_neigh2, b2.reshape(1, D),
      gamma2.reshape(1, D), beta2.reshape(1, D))


def kernel(features, edge_index, W_self1, W_neigh1, b1, gamma1, beta1,
           W_self2, W_neigh2, b2, gamma2, beta2):
    ei = edge_index.astype(jnp.int32)

    parts1, degp = _sc_aggregate(with_deg=True)(features, ei)
    h1 = _tc_mid(features, parts1, degp, W_self1, W_neigh1, b1,
                 gamma1, beta1)
    (parts2,) = _sc_aggregate(with_deg=False)(h1, ei)
    return _tc_final(h1, parts2, degp, W_self2, W_neigh2, b2,
                     gamma2, beta2)
